# baseline (speedup 1.0000x reference)
import functools

import jax
import jax.numpy as jnp
from jax import lax
from jax.experimental import pallas as pl
from jax.experimental.pallas import tpu as pltpu
from jax.experimental.pallas import tpu_sc as plsc

B, S, C = 32, 32768, 64
K = 8
LANES = 16
NGROUPS = C // LANES
CHUNK = 256
NCHUNKS = S // CHUNK
RBLK = 8
NBLKS = CHUNK // RBLK

_info = plsc.get_sparse_core_info()
NC, NS = _info.num_cores, _info.num_subcores


_SORT8 = [
    (0, 1), (2, 3), (4, 5), (6, 7),
    (0, 2), (1, 3), (4, 6), (5, 7),
    (1, 2), (5, 6),
    (0, 4), (1, 5), (2, 6), (3, 7),
    (2, 4), (3, 5),
    (1, 2), (3, 4), (5, 6),
]

_BITONIC8 = [
    (0, 4), (1, 5), (2, 6), (3, 7),
    (0, 2), (1, 3), (4, 6), (5, 7),
    (0, 1), (2, 3), (4, 5), (6, 7),
]


def _merge_block(state, vs):
    v = list(vs)
    for i, j in _SORT8:
        hi = jnp.maximum(v[i], v[j])
        lo = jnp.minimum(v[i], v[j])
        v[i], v[j] = hi, lo
    z = [jnp.maximum(state[i], v[K - 1 - i]) for i in range(K)]
    for i, j in _BITONIC8:
        hi = jnp.maximum(z[i], z[j])
        lo = jnp.minimum(z[i], z[j])
        z[i], z[j] = hi, lo
    return tuple(z)


def _treemax(vs):
    while len(vs) > 1:
        vs = [jnp.maximum(vs[i], vs[i + 1]) for i in range(0, len(vs) - 1, 2)] + (
            [vs[-1]] if len(vs) % 2 else []
        )
    return vs[0]


@functools.partial(
    pl.kernel,
    mesh=plsc.VectorSubcoreMesh(core_axis_name="c", subcore_axis_name="s"),
    out_type=jax.ShapeDtypeStruct((B, K, C), jnp.float32),
    compiler_params=pltpu.CompilerParams(needs_layout_passes=False),
    scratch_types=[
        pltpu.VMEM((CHUNK, C), jnp.float32),
        pltpu.VMEM((CHUNK, C), jnp.float32),
        pltpu.VMEM((K, C), jnp.float32),
        pltpu.SemaphoreType.DMA,
        pltpu.SemaphoreType.DMA,
    ],
)
def _topk_sc(x_hbm, out_hbm, buf0, buf1, outb, sem0, sem1):
    cid = lax.axis_index("c")
    sid = lax.axis_index("s")
    b = sid * NC + cid

    bufs = (buf0, buf1)
    sems = (sem0, sem1)

    def copy(ci, slot):
        return pltpu.make_async_copy(
            x_hbm.at[b, pl.ds(ci * CHUNK, CHUNK)], bufs[slot], sems[slot]
        )

    init = tuple(
        jnp.full((LANES,), -jnp.inf, jnp.float32) for _ in range(K * NGROUPS)
    )

    def screen_and_merge(buf, r0, g, sg):
        vs = [buf[r0 + r, pl.ds(g * LANES, LANES)] for r in range(RBLK)]
        return _merge_block(sg, vs)

    UNROLL = 2

    def process(buf, state):
        def blk_body(bi, st):
            r0 = bi * UNROLL * RBLK
            new_st = []
            for g in range(NGROUPS):
                sg = st[g * K:(g + 1) * K]
                for u in range(UNROLL):
                    sg = screen_and_merge(buf, r0 + u * RBLK, g, sg)
                new_st.extend(sg)
            return tuple(new_st)

        return lax.fori_loop(0, NBLKS // UNROLL, blk_body, state)

    copy(0, 0).start()

    def pair_body(i, state):
        ci = 2 * i
        copy(ci, 0).wait()
        copy(ci + 1, 1).start()
        state = process(buf0, state)
        copy(ci + 1, 1).wait()

        @pl.when(ci + 2 < NCHUNKS)
        def _():
            copy(ci + 2, 0).start()

        return process(buf1, state)

    final = lax.fori_loop(0, NCHUNKS // 2, pair_body, init)

    for g in range(NGROUPS):
        for j in range(K):
            outb[j, pl.ds(g * LANES, LANES)] = final[g * K + j]
    pltpu.sync_copy(outb, out_hbm.at[b])


def kernel(x):
    return _topk_sc(x)


# speedup vs baseline: 1.1810x; 1.1810x over previous
import functools

import jax
import jax.numpy as jnp
from jax import lax
from jax.experimental import pallas as pl
from jax.experimental.pallas import tpu as pltpu
from jax.experimental.pallas import tpu_sc as plsc

B, S, C = 32, 32768, 64
K = 8
LANES = 16
NGROUPS = C // LANES
CHUNK = 256
NCHUNKS = S // CHUNK
RBLK = 8
NBLKS = CHUNK // RBLK

_info = plsc.get_sparse_core_info()
NC, NS = _info.num_cores, _info.num_subcores


_SORT8 = [
    (0, 1), (2, 3), (4, 5), (6, 7),
    (0, 2), (1, 3), (4, 6), (5, 7),
    (1, 2), (5, 6),
    (0, 4), (1, 5), (2, 6), (3, 7),
    (2, 4), (3, 5),
    (1, 2), (3, 4), (5, 6),
]

_BITONIC8 = [
    (0, 4), (1, 5), (2, 6), (3, 7),
    (0, 2), (1, 3), (4, 6), (5, 7),
    (0, 1), (2, 3), (4, 5), (6, 7),
]


def _merge_block(state, vs):
    v = list(vs)
    for i, j in _SORT8:
        hi = jnp.maximum(v[i], v[j])
        lo = jnp.minimum(v[i], v[j])
        v[i], v[j] = hi, lo
    z = [jnp.maximum(state[i], v[K - 1 - i]) for i in range(K)]
    for i, j in _BITONIC8:
        hi = jnp.maximum(z[i], z[j])
        lo = jnp.minimum(z[i], z[j])
        z[i], z[j] = hi, lo
    return tuple(z)


def _treemax(vs):
    while len(vs) > 1:
        vs = [jnp.maximum(vs[i], vs[i + 1]) for i in range(0, len(vs) - 1, 2)] + (
            [vs[-1]] if len(vs) % 2 else []
        )
    return vs[0]


@functools.partial(
    pl.kernel,
    mesh=plsc.VectorSubcoreMesh(core_axis_name="c", subcore_axis_name="s"),
    out_type=jax.ShapeDtypeStruct((B, K, C), jnp.float32),
    compiler_params=pltpu.CompilerParams(needs_layout_passes=False),
    scratch_types=[
        pltpu.VMEM((CHUNK, C), jnp.float32),
        pltpu.VMEM((CHUNK, C), jnp.float32),
        pltpu.VMEM((K, C), jnp.float32),
        pltpu.SemaphoreType.DMA,
        pltpu.SemaphoreType.DMA,
    ],
)
def _topk_sc(x_hbm, out_hbm, buf0, buf1, outb, sem0, sem1):
    cid = lax.axis_index("c")
    sid = lax.axis_index("s")
    b = sid * NC + cid

    bufs = (buf0, buf1)
    sems = (sem0, sem1)

    def copy(ci, slot):
        return pltpu.make_async_copy(
            x_hbm.at[b, pl.ds(ci * CHUNK, CHUNK)], bufs[slot], sems[slot]
        )

    init = tuple(
        jnp.full((LANES,), -jnp.inf, jnp.float32) for _ in range(K * NGROUPS)
    )

    def screen_and_merge(buf, r0, g, sg):
        vs = [buf[r0 + r, pl.ds(g * LANES, LANES)] for r in range(RBLK)]
        bmax = _treemax(list(vs))
        cnt = plsc.all_reduce_population_count(bmax > sg[K - 1])
        pred = cnt[0] > 0

        def do(ops):
            return _merge_block(tuple(ops[:K]), tuple(ops[K:]))

        def skip(ops):
            return tuple(ops[:K])

        return lax.cond(pred, do, skip, tuple(sg) + tuple(vs))

    UNROLL = 2

    def process(buf, state):
        def blk_body(bi, st):
            r0 = bi * UNROLL * RBLK
            new_st = []
            for g in range(NGROUPS):
                sg = st[g * K:(g + 1) * K]
                for u in range(UNROLL):
                    sg = screen_and_merge(buf, r0 + u * RBLK, g, sg)
                new_st.extend(sg)
            return tuple(new_st)

        return lax.fori_loop(0, NBLKS // UNROLL, blk_body, state)

    copy(0, 0).start()

    def pair_body(i, state):
        ci = 2 * i
        copy(ci, 0).wait()
        copy(ci + 1, 1).start()
        state = process(buf0, state)
        copy(ci + 1, 1).wait()

        @pl.when(ci + 2 < NCHUNKS)
        def _():
            copy(ci + 2, 0).start()

        return process(buf1, state)

    final = lax.fori_loop(0, NCHUNKS // 2, pair_body, init)

    for g in range(NGROUPS):
        for j in range(K):
            outb[j, pl.ds(g * LANES, LANES)] = final[g * K + j]
    pltpu.sync_copy(outb, out_hbm.at[b])


def kernel(x):
    return _topk_sc(x)
